# Initial kernel scaffold; baseline (speedup 1.0000x reference)
import functools

import jax
import jax.numpy as jnp
from jax import lax
from jax.experimental import pallas as pl
from jax.experimental.pallas import tpu as pltpu
from jax.experimental.pallas import tpu_sc as plsc

_MAX_ORIGINAL_ID = 99999
_NUM_ORIGINAL = 100000
_EMBED_DIM = 1024
_N_TOKENS = 4 * 4096

_NUM_WORKERS = 32
_PER_WORKER = _N_TOKENS // _NUM_WORKERS
_CHUNK = 32
_NUM_CHUNKS = _PER_WORKER // _CHUNK
_LANES = 16


def _body(ids_hbm, w_hbm, aw_hbm, out_hbm,
          ids_v, idx0, idx1, rows0, rows1,
          gsem0, gsem1, psem0, psem1):
    nc = 2
    wid = lax.axis_index("s") * nc + lax.axis_index("c")
    base = wid * _PER_WORKER

    idxs = (idx0, idx1)
    rows = (rows0, rows1)
    gsems = (gsem0, gsem1)
    psems = (psem0, psem1)

    pltpu.sync_copy(ids_hbm.at[pl.ds(base, _PER_WORKER)], ids_v)

    def prep(c, b):
        for g in range(_CHUNK // _LANES):
            idv = ids_v[pl.ds(c * _CHUNK + g * _LANES, _LANES)]
            idxs[b][pl.ds(g * _LANES, _LANES)] = jnp.where(
                idv > _MAX_ORIGINAL_ID, 0, idv)

    def fire_gather(c, b):
        del c
        return pltpu.async_copy(w_hbm.at[idxs[b]], rows[b], gsems[b])

    def fixup(c, b):
        for g in range(_CHUNK // _LANES):
            idv = ids_v[pl.ds(c * _CHUNK + g * _LANES, _LANES)]
            mx = jnp.max(idv)

            @pl.when(mx > _MAX_ORIGINAL_ID)
            def _():
                def lane_body(lane, carry):
                    aid = jnp.max(jnp.where(
                        lax.iota(jnp.int32, _LANES) == lane, idv, -1))

                    @pl.when(aid > _MAX_ORIGINAL_ID)
                    def _():
                        pltpu.sync_copy(
                            aw_hbm.at[pl.ds(aid - _NUM_ORIGINAL, 1)],
                            rows[b].at[pl.ds(g * _LANES + lane, 1)])
                    return carry

                lax.fori_loop(0, _LANES, lane_body, 0)

    def fire_put(c, b):
        return pltpu.async_copy(
            rows[b], out_hbm.at[pl.ds(base + c * _CHUNK, _CHUNK)], psems[b])

    g_h = [None] * _NUM_CHUNKS
    p_h = [None] * _NUM_CHUNKS
    prep(0, 0)
    g_h[0] = fire_gather(0, 0)
    for c in range(_NUM_CHUNKS):
        b = c & 1
        nb = 1 - b
        if c + 1 < _NUM_CHUNKS:
            if c >= 1:
                p_h[c - 1].wait()
            prep(c + 1, nb)
            g_h[c + 1] = fire_gather(c + 1, nb)
        g_h[c].wait()
        fixup(c, b)
        p_h[c] = fire_put(c, b)
    p_h[_NUM_CHUNKS - 2].wait()
    p_h[_NUM_CHUNKS - 1].wait()


@jax.jit
def _run(ids_flat, weight, additional_weight):
    mesh = plsc.VectorSubcoreMesh(core_axis_name="c", subcore_axis_name="s")
    return pl.kernel(
        _body,
        out_type=jax.ShapeDtypeStruct((_N_TOKENS, _EMBED_DIM), jnp.float32),
        mesh=mesh,
        scratch_types=[
            pltpu.VMEM((_PER_WORKER,), jnp.int32),
            pltpu.VMEM((_CHUNK,), jnp.int32),
            pltpu.VMEM((_CHUNK,), jnp.int32),
            pltpu.VMEM((_CHUNK, _EMBED_DIM), jnp.float32),
            pltpu.VMEM((_CHUNK, _EMBED_DIM), jnp.float32),
            pltpu.SemaphoreType.DMA,
            pltpu.SemaphoreType.DMA,
            pltpu.SemaphoreType.DMA,
            pltpu.SemaphoreType.DMA,
        ],
    )(ids_flat, weight, additional_weight)


def kernel(input_ids, weight, additional_weight):
    batch, seq = input_ids.shape
    ids_flat = input_ids.reshape(-1)
    out = _run(ids_flat, weight, additional_weight)
    return out.reshape(batch, seq, weight.shape[1])


# speedup vs baseline: 2.4112x; 2.4112x over previous
import functools

import jax
import jax.numpy as jnp
from jax import lax
from jax.experimental import pallas as pl
from jax.experimental.pallas import tpu as pltpu
from jax.experimental.pallas import tpu_sc as plsc

_MAX_ORIGINAL_ID = 99999
_NUM_ORIGINAL = 100000
_EMBED_DIM = 1024
_N_TOKENS = 4 * 4096

_NUM_WORKERS = 32
_PER_WORKER = _N_TOKENS // _NUM_WORKERS
_CHUNK = 32
_NUM_CHUNKS = _PER_WORKER // _CHUNK
_LANES = 16


def _body(ids_hbm, w_hbm, aw_hbm, out_hbm,
          ids_v, idx0, idx1, rows0, rows1,
          gsem0, gsem1, psem0, psem1):
    nc = 2
    wid = lax.axis_index("s") * nc + lax.axis_index("c")
    base = wid * _PER_WORKER

    idxs = (idx0, idx1)
    rows = (rows0, rows1)
    gsems = (gsem0, gsem1)
    psems = (psem0, psem1)

    pltpu.sync_copy(ids_hbm.at[pl.ds(base, _PER_WORKER)], ids_v)

    def prep(c, b):
        for g in range(_CHUNK // _LANES):
            idv = ids_v[pl.ds(c * _CHUNK + g * _LANES, _LANES)]
            idxs[b][pl.ds(g * _LANES, _LANES)] = jnp.where(
                idv > _MAX_ORIGINAL_ID, 0, idv)

    def fire_gather(c, b):
        del c
        return pltpu.async_copy(w_hbm.at[idxs[b]], rows[b], gsems[b])

    def fixup(c, b):
        cnt = jnp.int32(0)
        for g in range(_CHUNK // _LANES):
            idv = ids_v[pl.ds(c * _CHUNK + g * _LANES, _LANES)]
            cnt += plsc.all_reduce_population_count(
                idv > _MAX_ORIGINAL_ID)[0]

        @pl.when(cnt > 0)
        def _():
            def lane_body(j, carry):
                splat = plsc.load_gather(
                    ids_v, [jnp.full((_LANES,), c * _CHUNK + j, jnp.int32)])
                aid = splat[0]

                @pl.when(aid > _MAX_ORIGINAL_ID)
                def _():
                    pltpu.sync_copy(
                        aw_hbm.at[pl.ds(aid - _NUM_ORIGINAL, 1)],
                        rows[b].at[pl.ds(j, 1)])
                return carry

            lax.fori_loop(0, _CHUNK, lane_body, 0)

    def fire_put(c, b):
        return pltpu.async_copy(
            rows[b], out_hbm.at[pl.ds(base + c * _CHUNK, _CHUNK)], psems[b])

    g_h = [None] * _NUM_CHUNKS
    p_h = [None] * _NUM_CHUNKS
    prep(0, 0)
    g_h[0] = fire_gather(0, 0)
    for c in range(_NUM_CHUNKS):
        b = c & 1
        nb = 1 - b
        if c + 1 < _NUM_CHUNKS:
            if c >= 1:
                p_h[c - 1].wait()
            prep(c + 1, nb)
            g_h[c + 1] = fire_gather(c + 1, nb)
        g_h[c].wait()
        fixup(c, b)
        p_h[c] = fire_put(c, b)
    p_h[_NUM_CHUNKS - 2].wait()
    p_h[_NUM_CHUNKS - 1].wait()


@jax.jit
def _run(ids_flat, weight, additional_weight):
    mesh = plsc.VectorSubcoreMesh(core_axis_name="c", subcore_axis_name="s")
    return pl.kernel(
        _body,
        out_type=jax.ShapeDtypeStruct((_N_TOKENS, _EMBED_DIM), jnp.float32),
        mesh=mesh,
        compiler_params=pltpu.CompilerParams(needs_layout_passes=False),
        scratch_types=[
            pltpu.VMEM((_PER_WORKER,), jnp.int32),
            pltpu.VMEM((_CHUNK,), jnp.int32),
            pltpu.VMEM((_CHUNK,), jnp.int32),
            pltpu.VMEM((_CHUNK, _EMBED_DIM), jnp.float32),
            pltpu.VMEM((_CHUNK, _EMBED_DIM), jnp.float32),
            pltpu.SemaphoreType.DMA,
            pltpu.SemaphoreType.DMA,
            pltpu.SemaphoreType.DMA,
            pltpu.SemaphoreType.DMA,
        ],
    )(ids_flat, weight, additional_weight)


def kernel(input_ids, weight, additional_weight):
    batch, seq = input_ids.shape
    ids_flat = input_ids.reshape(-1)
    out = _run(ids_flat, weight, additional_weight)
    return out.reshape(batch, seq, weight.shape[1])


# speedup vs baseline: 2.4490x; 1.0157x over previous
import functools

import jax
import jax.numpy as jnp
from jax import lax
from jax.experimental import pallas as pl
from jax.experimental.pallas import tpu as pltpu
from jax.experimental.pallas import tpu_sc as plsc

_MAX_ORIGINAL_ID = 99999
_NUM_ORIGINAL = 100000
_EMBED_DIM = 1024
_N_TOKENS = 4 * 4096

_NUM_WORKERS = 32
_PER_WORKER = _N_TOKENS // _NUM_WORKERS
_CHUNK = 32
_NUM_CHUNKS = _PER_WORKER // _CHUNK
_LANES = 16


def _body(ids_hbm, w_hbm, aw_hbm, out_hbm,
          ids_v, idx0, idx1, idx2, rows0, rows1, rows2,
          gsem0, gsem1, gsem2, psem0, psem1, psem2):
    nc = 2
    wid = lax.axis_index("s") * nc + lax.axis_index("c")
    base = wid * _PER_WORKER

    idxs = (idx0, idx1, idx2)
    rows = (rows0, rows1, rows2)
    gsems = (gsem0, gsem1, gsem2)
    psems = (psem0, psem1, psem2)

    pltpu.sync_copy(ids_hbm.at[pl.ds(base, _PER_WORKER)], ids_v)

    def prep(c, b):
        for g in range(_CHUNK // _LANES):
            idv = ids_v[pl.ds(c * _CHUNK + g * _LANES, _LANES)]
            idxs[b][pl.ds(g * _LANES, _LANES)] = jnp.where(
                idv > _MAX_ORIGINAL_ID, 0, idv)

    def fire_gather(c, b):
        del c
        return pltpu.async_copy(w_hbm.at[idxs[b]], rows[b], gsems[b])

    def fixup(c, b):
        cnt = jnp.int32(0)
        for g in range(_CHUNK // _LANES):
            idv = ids_v[pl.ds(c * _CHUNK + g * _LANES, _LANES)]
            cnt += plsc.all_reduce_population_count(
                idv > _MAX_ORIGINAL_ID)[0]

        @pl.when(cnt > 0)
        def _():
            def lane_body(j, carry):
                splat = plsc.load_gather(
                    ids_v, [jnp.full((_LANES,), c * _CHUNK + j, jnp.int32)])
                aid = splat[0]

                @pl.when(aid > _MAX_ORIGINAL_ID)
                def _():
                    pltpu.sync_copy(
                        aw_hbm.at[pl.ds(aid - _NUM_ORIGINAL, 1)],
                        rows[b].at[pl.ds(j, 1)])
                return carry

            lax.fori_loop(0, _CHUNK, lane_body, 0)

    def fire_put(c, b):
        return pltpu.async_copy(
            rows[b], out_hbm.at[pl.ds(base + c * _CHUNK, _CHUNK)], psems[b])

    nbuf = 3
    g_h = [None] * _NUM_CHUNKS
    p_h = [None] * _NUM_CHUNKS
    prep(0, 0)
    g_h[0] = fire_gather(0, 0)
    for c in range(_NUM_CHUNKS):
        b = c % nbuf
        if c + 1 < _NUM_CHUNKS:
            nb = (c + 1) % nbuf
            if c >= 2:
                p_h[c - 2].wait()
            prep(c + 1, nb)
            g_h[c + 1] = fire_gather(c + 1, nb)
        g_h[c].wait()
        fixup(c, b)
        p_h[c] = fire_put(c, b)
    for c in range(max(0, _NUM_CHUNKS - 3), _NUM_CHUNKS):
        p_h[c].wait()


@jax.jit
def _run(ids_flat, weight, additional_weight):
    mesh = plsc.VectorSubcoreMesh(core_axis_name="c", subcore_axis_name="s")
    return pl.kernel(
        _body,
        out_type=jax.ShapeDtypeStruct((_N_TOKENS, _EMBED_DIM), jnp.float32),
        mesh=mesh,
        compiler_params=pltpu.CompilerParams(needs_layout_passes=False),
        scratch_types=[
            pltpu.VMEM((_PER_WORKER,), jnp.int32),
            pltpu.VMEM((_CHUNK,), jnp.int32),
            pltpu.VMEM((_CHUNK,), jnp.int32),
            pltpu.VMEM((_CHUNK,), jnp.int32),
            pltpu.VMEM((_CHUNK, _EMBED_DIM), jnp.float32),
            pltpu.VMEM((_CHUNK, _EMBED_DIM), jnp.float32),
            pltpu.VMEM((_CHUNK, _EMBED_DIM), jnp.float32),
            pltpu.SemaphoreType.DMA,
            pltpu.SemaphoreType.DMA,
            pltpu.SemaphoreType.DMA,
            pltpu.SemaphoreType.DMA,
            pltpu.SemaphoreType.DMA,
            pltpu.SemaphoreType.DMA,
        ],
    )(ids_flat, weight, additional_weight)


def kernel(input_ids, weight, additional_weight):
    batch, seq = input_ids.shape
    ids_flat = input_ids.reshape(-1)
    out = _run(ids_flat, weight, additional_weight)
    return out.reshape(batch, seq, weight.shape[1])


# speedup vs baseline: 2.4506x; 1.0007x over previous
import functools

import jax
import jax.numpy as jnp
from jax import lax
from jax.experimental import pallas as pl
from jax.experimental.pallas import tpu as pltpu
from jax.experimental.pallas import tpu_sc as plsc

_MAX_ORIGINAL_ID = 99999
_NUM_ORIGINAL = 100000
_EMBED_DIM = 1024
_N_TOKENS = 4 * 4096

_NUM_WORKERS = 32
_PER_WORKER = _N_TOKENS // _NUM_WORKERS
_CHUNK = 32
_NUM_CHUNKS = _PER_WORKER // _CHUNK
_LANES = 16


def _body(ids_hbm, w_hbm, aw_hbm, out_hbm,
          ids_v, idx0, idx1, idx2, rows0, rows1, rows2,
          gsem0, gsem1, gsem2, psem0, psem1, psem2):
    nc = 2
    wid = lax.axis_index("s") * nc + lax.axis_index("c")
    wpb = 4096 // _PER_WORKER
    bat = wid // wpb
    sbase = (wid % wpb) * _PER_WORKER

    idxs = (idx0, idx1, idx2)
    rows = (rows0, rows1, rows2)
    gsems = (gsem0, gsem1, gsem2)
    psems = (psem0, psem1, psem2)

    pltpu.sync_copy(ids_hbm.at[bat, pl.ds(sbase, _PER_WORKER)], ids_v)

    def prep(c, b):
        for g in range(_CHUNK // _LANES):
            idv = ids_v[pl.ds(c * _CHUNK + g * _LANES, _LANES)]
            idxs[b][pl.ds(g * _LANES, _LANES)] = jnp.where(
                idv > _MAX_ORIGINAL_ID, 0, idv)

    def fire_gather(c, b):
        del c
        return pltpu.async_copy(w_hbm.at[idxs[b]], rows[b], gsems[b])

    def fixup(c, b):
        cnt = jnp.int32(0)
        for g in range(_CHUNK // _LANES):
            idv = ids_v[pl.ds(c * _CHUNK + g * _LANES, _LANES)]
            cnt += plsc.all_reduce_population_count(
                idv > _MAX_ORIGINAL_ID)[0]

        @pl.when(cnt > 0)
        def _():
            def lane_body(j, carry):
                splat = plsc.load_gather(
                    ids_v, [jnp.full((_LANES,), c * _CHUNK + j, jnp.int32)])
                aid = splat[0]

                @pl.when(aid > _MAX_ORIGINAL_ID)
                def _():
                    pltpu.sync_copy(
                        aw_hbm.at[pl.ds(aid - _NUM_ORIGINAL, 1)],
                        rows[b].at[pl.ds(j, 1)])
                return carry

            lax.fori_loop(0, _CHUNK, lane_body, 0)

    def fire_put(c, b):
        return pltpu.async_copy(
            rows[b], out_hbm.at[bat, pl.ds(sbase + c * _CHUNK, _CHUNK)],
            psems[b])

    nbuf = 3
    g_h = [None] * _NUM_CHUNKS
    p_h = [None] * _NUM_CHUNKS
    prep(0, 0)
    g_h[0] = fire_gather(0, 0)
    for c in range(_NUM_CHUNKS):
        b = c % nbuf
        if c + 1 < _NUM_CHUNKS:
            nb = (c + 1) % nbuf
            if c >= 2:
                p_h[c - 2].wait()
            prep(c + 1, nb)
            g_h[c + 1] = fire_gather(c + 1, nb)
        g_h[c].wait()
        fixup(c, b)
        p_h[c] = fire_put(c, b)
    for c in range(max(0, _NUM_CHUNKS - 3), _NUM_CHUNKS):
        p_h[c].wait()


@jax.jit
def _run(ids, weight, additional_weight):
    mesh = plsc.VectorSubcoreMesh(core_axis_name="c", subcore_axis_name="s")
    return pl.kernel(
        _body,
        out_type=jax.ShapeDtypeStruct((4, 4096, _EMBED_DIM), jnp.float32),
        mesh=mesh,
        compiler_params=pltpu.CompilerParams(needs_layout_passes=False),
        scratch_types=[
            pltpu.VMEM((_PER_WORKER,), jnp.int32),
            pltpu.VMEM((_CHUNK,), jnp.int32),
            pltpu.VMEM((_CHUNK,), jnp.int32),
            pltpu.VMEM((_CHUNK,), jnp.int32),
            pltpu.VMEM((_CHUNK, _EMBED_DIM), jnp.float32),
            pltpu.VMEM((_CHUNK, _EMBED_DIM), jnp.float32),
            pltpu.VMEM((_CHUNK, _EMBED_DIM), jnp.float32),
            pltpu.SemaphoreType.DMA,
            pltpu.SemaphoreType.DMA,
            pltpu.SemaphoreType.DMA,
            pltpu.SemaphoreType.DMA,
            pltpu.SemaphoreType.DMA,
            pltpu.SemaphoreType.DMA,
        ],
    )(ids, weight, additional_weight)


def kernel(input_ids, weight, additional_weight):
    return _run(input_ids, weight, additional_weight)
